# Initial kernel scaffold; baseline (speedup 1.0000x reference)
import jax
import jax.numpy as jnp
from jax.experimental import pallas as pl


def kernel(instrument_ids, embedding_table):
    raise NotImplementedError("write your pallas kernel here")


# speedup vs baseline: 1.7955x; 1.7955x over previous
import jax
import jax.numpy as jnp
from jax import lax
from jax.experimental import pallas as pl
from jax.experimental.pallas import tpu as pltpu
from jax.experimental.pallas import tpu_sc as plsc

_VOCAB = 1000000
_DIM = 64
_BATCH = 16384
_HIST = 50
_TOTAL = _BATCH * _HIST
_NC = 2
_NS = 16
_NW = _NC * _NS
_PER_W = _TOTAL // _NW
_CHUNK = 512
_N_CHUNKS = _PER_W // _CHUNK


def _gather_body(table_hbm, idx_hbm, out_hbm, idx_v, rows_v, sem):
    wid = lax.axis_index("s") * _NC + lax.axis_index("c")
    base = wid * _PER_W

    def body(g, carry):
        off = pl.multiple_of(base + g * _CHUNK, _CHUNK)
        pltpu.sync_copy(idx_hbm.at[pl.ds(off, _CHUNK)], idx_v)
        pltpu.async_copy(table_hbm.at[idx_v], rows_v, sem).wait()
        pltpu.sync_copy(rows_v, out_hbm.at[pl.ds(off, _CHUNK)])
        return carry

    lax.fori_loop(0, _N_CHUNKS, body, 0)


def kernel(instrument_ids, embedding_table):
    idx_flat = instrument_ids.reshape(_TOTAL)
    mesh = plsc.VectorSubcoreMesh(core_axis_name="c", subcore_axis_name="s")
    out = pl.kernel(
        _gather_body,
        out_type=jax.ShapeDtypeStruct((_TOTAL, _DIM), jnp.float32),
        mesh=mesh,
        scratch_types=[
            pltpu.VMEM((_CHUNK,), jnp.int32),
            pltpu.VMEM((_CHUNK, _DIM), jnp.float32),
            pltpu.SemaphoreType.DMA,
        ],
        compiler_params=pltpu.CompilerParams(use_tc_tiling_on_sc=False),
    )(embedding_table, idx_flat)
    return out.reshape(_BATCH, _HIST, _DIM)


# speedup vs baseline: 1.8729x; 1.0431x over previous
import jax
import jax.numpy as jnp
from jax import lax
from jax.experimental import pallas as pl
from jax.experimental.pallas import tpu as pltpu
from jax.experimental.pallas import tpu_sc as plsc

_VOCAB = 1000000
_DIM = 64
_BATCH = 16384
_HIST = 50
_TOTAL = _BATCH * _HIST
_NC = 2
_NS = 16
_NW = _NC * _NS
_PER_W = _TOTAL // _NW
_CHUNK = 800
_N_CHUNKS = _PER_W // _CHUNK
_NBUF = 2


def _gather_body(table_hbm, idx_hbm, out_hbm,
                 idx0, idx1, rows0, rows1,
                 sem_i0, sem_i1, sem_g0, sem_g1, sem_s0, sem_s1):
    idx_v = (idx0, idx1)
    rows_v = (rows0, rows1)
    sem_i = (sem_i0, sem_i1)
    sem_g = (sem_g0, sem_g1)
    sem_s = (sem_s0, sem_s1)

    wid = lax.axis_index("s") * _NC + lax.axis_index("c")
    base = wid * _PER_W

    def chunk_off(c):
        return pl.multiple_of(base + c * _CHUNK, 8)

    for b in range(_NBUF):
        pltpu.async_copy(idx_hbm.at[pl.ds(chunk_off(b), _CHUNK)],
                         idx_v[b], sem_i[b])

    def step(g, carry):
        for b in range(_NBUF):
            c = g * _NBUF + b

            @pl.when(c >= _NBUF)
            def _():
                pltpu.make_async_copy(
                    rows_v[b],
                    out_hbm.at[pl.ds(chunk_off(c - _NBUF), _CHUNK)],
                    sem_s[b]).wait()

            pltpu.make_async_copy(
                idx_hbm.at[pl.ds(chunk_off(c), _CHUNK)],
                idx_v[b], sem_i[b]).wait()

            pltpu.async_copy(table_hbm.at[idx_v[b]], rows_v[b],
                             sem_g[b]).wait()

            @pl.when(c + _NBUF < _N_CHUNKS)
            def _():
                pltpu.async_copy(
                    idx_hbm.at[pl.ds(chunk_off(c + _NBUF), _CHUNK)],
                    idx_v[b], sem_i[b])

            pltpu.async_copy(rows_v[b],
                             out_hbm.at[pl.ds(chunk_off(c), _CHUNK)],
                             sem_s[b])
        return carry

    lax.fori_loop(0, _N_CHUNKS // _NBUF, step, 0)

    for b in range(_NBUF):
        c_last = _N_CHUNKS - _NBUF + b
        pltpu.make_async_copy(rows_v[b],
                              out_hbm.at[pl.ds(chunk_off(c_last), _CHUNK)],
                              sem_s[b]).wait()


def kernel(instrument_ids, embedding_table):
    idx_flat = instrument_ids.reshape(_TOTAL)
    mesh = plsc.VectorSubcoreMesh(core_axis_name="c", subcore_axis_name="s")
    out = pl.kernel(
        _gather_body,
        out_type=jax.ShapeDtypeStruct((_TOTAL, _DIM), jnp.float32),
        mesh=mesh,
        scratch_types=[
            pltpu.VMEM((_CHUNK,), jnp.int32),
            pltpu.VMEM((_CHUNK,), jnp.int32),
            pltpu.VMEM((_CHUNK, _DIM), jnp.float32),
            pltpu.VMEM((_CHUNK, _DIM), jnp.float32),
            pltpu.SemaphoreType.DMA,
            pltpu.SemaphoreType.DMA,
            pltpu.SemaphoreType.DMA,
            pltpu.SemaphoreType.DMA,
            pltpu.SemaphoreType.DMA,
            pltpu.SemaphoreType.DMA,
        ],
        compiler_params=pltpu.CompilerParams(use_tc_tiling_on_sc=False),
    )(embedding_table, idx_flat)
    return out.reshape(_BATCH, _HIST, _DIM)
